# baseline (speedup 1.0000x reference)
import functools

import jax
import jax.numpy as jnp
from jax import lax
from jax.experimental import pallas as pl
from jax.experimental.pallas import tpu as pltpu
from jax.experimental.pallas import tpu_sc as plsc

NC = 2
NS = 16
NW = NC * NS
LANES = 16
CHUNK = 32
NBUF = 3


def _build_gather(B, S, D, K, with_mask):
    N = B * K
    rows_pw = N // NW
    jpw = rows_pw // CHUNK
    workers_per_batch = NW // B

    mesh = plsc.VectorSubcoreMesh(core_axis_name="c", subcore_axis_name="s")

    out_type = [jax.ShapeDtypeStruct((N, D), jnp.float32)]
    if with_mask:
        out_type.append(jax.ShapeDtypeStruct((N,), jnp.int32))

    @functools.partial(
        pl.kernel,
        out_type=out_type,
        mesh=mesh,
        scratch_types=(
            [
                pltpu.VMEM((rows_pw,), jnp.int32),
                pltpu.VMEM((rows_pw,), jnp.int32),
            ]
            + [pltpu.VMEM((CHUNK, D), jnp.float32) for _ in range(NBUF)]
            + [pltpu.SemaphoreType.DMA for _ in range(2 * NBUF)]
        ),
    )
    def gather_kernel(table_hbm, idx_hbm, *rest):
        if with_mask:
            rows_out, mask_out = rest[0], rest[1]
            rest = rest[2:]
        else:
            rows_out = rest[0]
            rest = rest[1:]
        iv, m_v = rest[0], rest[1]
        bufs = rest[2:2 + NBUF]
        gsems = rest[2 + NBUF:2 + 2 * NBUF]
        wsems = rest[2 + 2 * NBUF:]

        cid = lax.axis_index("c")
        sid = lax.axis_index("s")
        wid = sid * NC + cid
        row0 = wid * rows_pw
        pltpu.sync_copy(idx_hbm.at[pl.ds(row0, rows_pw)], iv)
        batch_base = (wid // workers_per_batch) * S
        bb = jnp.full((LANES,), batch_base, dtype=jnp.int32)
        zero = jnp.zeros((LANES,), jnp.int32)
        one = jnp.ones((LANES,), jnp.int32)
        for t in range(rows_pw // LANES):
            sl = pl.ds(t * LANES, LANES)
            v = iv[sl]
            if with_mask:
                m_v[sl] = jnp.where(v > zero, one, zero)
            iv[sl] = v + bb
        if with_mask:
            pltpu.sync_copy(m_v, mask_out.at[pl.ds(row0, rows_pw)])

        def start_gather(j):
            return pltpu.async_copy(
                table_hbm.at[iv.at[pl.ds(j * CHUNK, CHUNK)]], bufs[j % NBUF],
                gsems[j % NBUF])

        gh = [None] * jpw
        wh = [None] * jpw
        for j in range(min(NBUF, jpw)):
            gh[j] = start_gather(j)
        for j in range(jpw):
            gh[j].wait()
            wh[j] = pltpu.async_copy(
                bufs[j % NBUF], rows_out.at[pl.ds(row0 + j * CHUNK, CHUNK)],
                wsems[j % NBUF])
            nxt = j + NBUF
            if nxt < jpw:
                wh[j].wait()
                gh[nxt] = start_gather(nxt)
        for j in range(max(0, jpw - NBUF), jpw):
            if j + NBUF >= jpw:
                wh[j].wait()

    return gather_kernel


def kernel(student_results, teacher_results, a_selected_indices,
           b_selected_indices):
    B, S, D = student_results.shape
    K = a_selected_indices.shape[1]
    student_flat = student_results.reshape(B * S, D)
    teacher_flat = teacher_results.reshape(B * S, D)
    a_idx = a_selected_indices.astype(jnp.int32).reshape(B * K)
    b_idx = b_selected_indices.astype(jnp.int32).reshape(B * K)
    a_rows, mask_i32 = _build_gather(B, S, D, K, True)(teacher_flat, a_idx)
    (b_rows,) = _build_gather(B, S, D, K, False)(student_flat, b_idx)
    return (b_rows.reshape(B, K, D),
            a_rows.reshape(B, K, D),
            mask_i32.reshape(B, K).astype(jnp.bool_))


# speedup vs baseline: 1.0859x; 1.0859x over previous
import functools

import jax
import jax.numpy as jnp
from jax import lax
from jax.experimental import pallas as pl
from jax.experimental.pallas import tpu as pltpu
from jax.experimental.pallas import tpu_sc as plsc

NC = 2
NS = 16
NW = NC * NS
LANES = 16
CHUNK = 32
NBUF = 3


def _build_gather(B, S, D, K):
    N = B * K
    rows_pw = N // NW
    jpw = rows_pw // CHUNK
    workers_per_batch = NW // B
    kpw = K // workers_per_batch

    mesh = plsc.VectorSubcoreMesh(core_axis_name="c", subcore_axis_name="s")

    @functools.partial(
        pl.kernel,
        out_type=[
            jax.ShapeDtypeStruct((B, K, D), jnp.float32),
            jax.ShapeDtypeStruct((B, K, D), jnp.float32),
            jax.ShapeDtypeStruct((B, K), jnp.int32),
        ],
        mesh=mesh,
        scratch_types=(
            [
                pltpu.VMEM((rows_pw,), jnp.int32),
                pltpu.VMEM((rows_pw,), jnp.int32),
                pltpu.VMEM((rows_pw,), jnp.int32),
            ]
            + [pltpu.VMEM((CHUNK, D), jnp.float32) for _ in range(NBUF)]
            + [pltpu.SemaphoreType.DMA for _ in range(2 * NBUF)]
        ),
    )
    def gather_kernel(student_hbm, teacher_hbm, a_idx_hbm, b_idx_hbm,
                      b_out, a_out, mask_out, a_iv, b_iv, m_v,
                      *bufs_and_sems):
        bufs = bufs_and_sems[:NBUF]
        gsems = bufs_and_sems[NBUF:2 * NBUF]
        wsems = bufs_and_sems[2 * NBUF:]

        cid = lax.axis_index("c")
        sid = lax.axis_index("s")
        wid = sid * NC + cid
        batch = wid // workers_per_batch
        k0 = (wid % workers_per_batch) * kpw
        pltpu.sync_copy(a_idx_hbm.at[batch, pl.ds(k0, kpw)], a_iv)
        pltpu.sync_copy(b_idx_hbm.at[batch, pl.ds(k0, kpw)], b_iv)
        zero = jnp.zeros((LANES,), jnp.int32)
        one = jnp.ones((LANES,), jnp.int32)
        for t in range(rows_pw // LANES):
            sl = pl.ds(t * LANES, LANES)
            m_v[sl] = jnp.where(a_iv[sl] > zero, one, zero)
        pltpu.sync_copy(m_v, mask_out.at[batch, pl.ds(k0, kpw)])

        jobs = [(teacher_hbm, a_iv, a_out, j) for j in range(jpw)]
        jobs += [(student_hbm, b_iv, b_out, j) for j in range(jpw)]
        nj = len(jobs)

        def start_gather(i):
            table, iv, _, j = jobs[i]
            return pltpu.async_copy(
                table.at[batch].at[iv.at[pl.ds(j * CHUNK, CHUNK)]],
                bufs[i % NBUF], gsems[i % NBUF])

        gh = [None] * nj
        wh = [None] * nj
        for i in range(min(NBUF, nj)):
            gh[i] = start_gather(i)
        for i in range(nj):
            gh[i].wait()
            _, _, out, j = jobs[i]
            wh[i] = pltpu.async_copy(
                bufs[i % NBUF],
                out.at[batch, pl.ds(k0 + j * CHUNK, CHUNK)],
                wsems[i % NBUF])
            nxt = i + NBUF
            if nxt < nj:
                wh[i].wait()
                gh[nxt] = start_gather(nxt)
        for i in range(max(0, nj - NBUF), nj):
            wh[i].wait()

    return gather_kernel


def kernel(student_results, teacher_results, a_selected_indices,
           b_selected_indices):
    B, S, D = student_results.shape
    K = a_selected_indices.shape[1]
    a_idx = a_selected_indices.astype(jnp.int32)
    b_idx = b_selected_indices.astype(jnp.int32)
    b_rows, a_rows, mask_i32 = _build_gather(B, S, D, K)(
        student_results, teacher_results, a_idx, b_idx)
    return (b_rows, a_rows, mask_i32.astype(jnp.bool_))
